# Initial kernel scaffold; baseline (speedup 1.0000x reference)
import jax
import jax.numpy as jnp
from jax.experimental import pallas as pl


def kernel(x, emb0_w, emb1_w, g4, b4, g7, b7, g10, b10, g13, b13, g16, b16, fcW, fcb):
    raise NotImplementedError("write your pallas kernel here")


# speedup vs baseline: 12.8209x; 12.8209x over previous
import functools

import jax
import jax.numpy as jnp
from jax import lax
from jax.experimental import pallas as pl
from jax.experimental.pallas import tpu as pltpu
from jax.experimental.pallas import tpu_sc as plsc

B, L, V, D, OUT = 4096, 200, 30000, 512, 64
EPS = 1e-5


VB = 600


def _table_body(e0, e1, g4, b4, g7, b7, g10, b10, g13, b13, g16, b16,
                fcw, fcb, out):
    w = e0[...] + e1[...]
    for g, b in ((g4, b4), (g7, b7), (g10, b10), (g13, b13), (g16, b16)):
        mu = jnp.mean(w, axis=1, keepdims=True)
        var = jnp.mean((w - mu) ** 2, axis=1, keepdims=True)
        w = (w - mu) * lax.rsqrt(var + EPS) * g[...] + b[...]
    acc = lax.dot_general(w, fcw[...], (((1,), (1,)), ((), ())),
                          preferred_element_type=jnp.float32)
    out[...] = acc + fcb[...]


def _build_table(e0, e1, lns, fcw, fcb):
    full = lambda i: (0, 0)
    in_specs = [
        pl.BlockSpec((VB, D), lambda i: (i, 0)),
        pl.BlockSpec((VB, D), lambda i: (i, 0)),
    ]
    in_specs += [pl.BlockSpec((1, D), full) for _ in range(10)]
    in_specs += [pl.BlockSpec((OUT, D), full), pl.BlockSpec((1, OUT), full)]
    return pl.pallas_call(
        _table_body,
        grid=(V // VB,),
        in_specs=in_specs,
        out_specs=pl.BlockSpec((VB, OUT), lambda i: (i, 0)),
        out_shape=jax.ShapeDtypeStruct((V, OUT), jnp.float32),
    )(e0, e1, *lns, fcw, fcb)


NC, NS = 2, 16
NW = NC * NS
NTOK = B * L
CHUNK = 128
NCHUNK = NTOK // (NW * CHUNK)

def _gather_body(table_hbm, idx_hbm, out_hbm, idx_v, rows_v, sem):
    wid = lax.axis_index("s") * NC + lax.axis_index("c")
    pltpu.sync_copy(idx_hbm.at[wid], idx_v)

    @pl.loop(0, NCHUNK)
    def _chunk(c):
        pltpu.async_copy(table_hbm.at[idx_v.at[c]], rows_v, sem).wait()
        pltpu.sync_copy(rows_v, out_hbm.at[wid, c])


@functools.lru_cache(maxsize=None)
def _get_gather():
    mesh = plsc.VectorSubcoreMesh(core_axis_name="c", subcore_axis_name="s",
                                  num_cores=NC, num_subcores=NS)
    return pl.kernel(
        _gather_body,
        out_type=jax.ShapeDtypeStruct((NW, NCHUNK, CHUNK, OUT), jnp.float32),
        mesh=mesh,
        scratch_types=[
            pltpu.VMEM((NCHUNK, CHUNK), jnp.int32),
            pltpu.VMEM((CHUNK, OUT), jnp.float32),
            pltpu.SemaphoreType.DMA,
        ],
        compiler_params=pltpu.CompilerParams(use_tc_tiling_on_sc=False),
    )


def kernel(x, emb0_w, emb1_w, g4, b4, g7, b7, g10, b10, g13, b13, g16, b16,
           fcW, fcb):
    lns = [a.reshape(1, D) for a in (g4, b4, g7, b7, g10, b10, g13, b13,
                                     g16, b16)]
    table = _build_table(emb0_w, emb1_w, lns, fcW, fcb.reshape(1, OUT))
    idx = x.astype(jnp.int32).reshape(NW, NCHUNK, CHUNK)
    out = _get_gather()(table, idx)
    return out.reshape(B, L, OUT)


# speedup vs baseline: 15.0500x; 1.1739x over previous
import functools

import jax
import jax.numpy as jnp
from jax import lax
from jax.experimental import pallas as pl
from jax.experimental.pallas import tpu as pltpu
from jax.experimental.pallas import tpu_sc as plsc

B, L, V, D, OUT = 4096, 200, 30000, 512, 64
EPS = 1e-5


VB = 600


def _table_body(e0, e1, g4, b4, g7, b7, g10, b10, g13, b13, g16, b16,
                fcw, fcb, out):
    w = e0[...] + e1[...]
    for g, b in ((g4, b4), (g7, b7), (g10, b10), (g13, b13), (g16, b16)):
        mu = jnp.mean(w, axis=1, keepdims=True)
        var = jnp.mean((w - mu) ** 2, axis=1, keepdims=True)
        w = (w - mu) * lax.rsqrt(var + EPS) * g[...] + b[...]
    acc = lax.dot_general(w, fcw[...], (((1,), (1,)), ((), ())),
                          preferred_element_type=jnp.float32)
    out[...] = acc + fcb[...]


def _build_table(e0, e1, lns, fcw, fcb):
    full = lambda i: (0, 0)
    in_specs = [
        pl.BlockSpec((VB, D), lambda i: (i, 0)),
        pl.BlockSpec((VB, D), lambda i: (i, 0)),
    ]
    in_specs += [pl.BlockSpec((1, D), full) for _ in range(10)]
    in_specs += [pl.BlockSpec((OUT, D), full), pl.BlockSpec((1, OUT), full)]
    return pl.pallas_call(
        _table_body,
        grid=(V // VB,),
        in_specs=in_specs,
        out_specs=pl.BlockSpec((VB, OUT), lambda i: (i, 0)),
        out_shape=jax.ShapeDtypeStruct((V, OUT), jnp.float32),
    )(e0, e1, *lns, fcw, fcb)


NC, NS = 2, 16
NW = NC * NS
NROW = B // NW
C0, C1 = 128, L - 128


def _gather_body(table_hbm, x_hbm, out_hbm, idx_v, rows_v, sg0, sg1):
    wid = lax.axis_index("s") * NC + lax.axis_index("c")
    r0 = wid * NROW
    pltpu.sync_copy(x_hbm.at[pl.ds(r0, NROW)], idx_v)
    sgs = (sg0, sg1)

    def fire(r, bf):
        pltpu.async_copy(table_hbm.at[idx_v.at[r, pl.ds(0, C0)]],
                         rows_v.at[bf, pl.ds(0, C0)], sgs[bf])
        pltpu.async_copy(table_hbm.at[idx_v.at[r, pl.ds(C0, C1)]],
                         rows_v.at[bf, pl.ds(C0, C1)], sgs[bf])

    def wait(r, bf):
        pltpu.make_async_copy(table_hbm.at[idx_v.at[r, pl.ds(0, C0)]],
                              rows_v.at[bf, pl.ds(0, C0)], sgs[bf]).wait()
        pltpu.make_async_copy(table_hbm.at[idx_v.at[r, pl.ds(C0, C1)]],
                              rows_v.at[bf, pl.ds(C0, C1)], sgs[bf]).wait()

    def store(r, bf):
        pltpu.sync_copy(rows_v.at[bf], out_hbm.at[r0 + r])

    fire(0, 0)

    @pl.loop(0, NROW // 2 - 1)
    def _pair(ro):
        r = ro * 2
        fire(r + 1, 1)
        wait(r, 0)
        store(r, 0)
        fire(r + 2, 0)
        wait(r + 1, 1)
        store(r + 1, 1)

    fire(NROW - 1, 1)
    wait(NROW - 2, 0)
    store(NROW - 2, 0)
    wait(NROW - 1, 1)
    store(NROW - 1, 1)


@functools.lru_cache(maxsize=None)
def _get_gather():
    mesh = plsc.VectorSubcoreMesh(core_axis_name="c", subcore_axis_name="s",
                                  num_cores=NC, num_subcores=NS)
    return pl.kernel(
        _gather_body,
        out_type=jax.ShapeDtypeStruct((B, L, OUT), jnp.float32),
        mesh=mesh,
        scratch_types=[
            pltpu.VMEM((NROW, L), jnp.int32),
            pltpu.VMEM((2, L, OUT), jnp.float32),
            pltpu.SemaphoreType.DMA,
            pltpu.SemaphoreType.DMA,
        ],
        compiler_params=pltpu.CompilerParams(use_tc_tiling_on_sc=False),
    )


def kernel(x, emb0_w, emb1_w, g4, b4, g7, b7, g10, b10, g13, b13, g16, b16,
           fcW, fcb):
    lns = [a.reshape(1, D) for a in (g4, b4, g7, b7, g10, b10, g13, b13,
                                     g16, b16)]
    table = _build_table(emb0_w, emb1_w, lns, fcW, fcb.reshape(1, OUT))
    return _get_gather()(table, x.astype(jnp.int32))


# speedup vs baseline: 15.0535x; 1.0002x over previous
import functools

import jax
import jax.numpy as jnp
from jax import lax
from jax.experimental import pallas as pl
from jax.experimental.pallas import tpu as pltpu
from jax.experimental.pallas import tpu_sc as plsc

B, L, V, D, OUT = 4096, 200, 30000, 512, 64
EPS = 1e-5


VB = 600


def _table_body(e0, e1, g4, b4, g7, b7, g10, b10, g13, b13, g16, b16,
                fcw, fcb, out):
    w = e0[...] + e1[...]
    for g, b in ((g4, b4), (g7, b7), (g10, b10), (g13, b13), (g16, b16)):
        mu = jnp.mean(w, axis=1, keepdims=True)
        var = jnp.mean((w - mu) ** 2, axis=1, keepdims=True)
        w = (w - mu) * lax.rsqrt(var + EPS) * g[...] + b[...]
    acc = lax.dot_general(w, fcw[...], (((1,), (1,)), ((), ())),
                          preferred_element_type=jnp.float32)
    out[...] = acc + fcb[...]


def _build_table(e0, e1, lns, fcw, fcb):
    full = lambda i: (0, 0)
    in_specs = [
        pl.BlockSpec((VB, D), lambda i: (i, 0)),
        pl.BlockSpec((VB, D), lambda i: (i, 0)),
    ]
    in_specs += [pl.BlockSpec((1, D), full) for _ in range(10)]
    in_specs += [pl.BlockSpec((OUT, D), full), pl.BlockSpec((1, OUT), full)]
    return pl.pallas_call(
        _table_body,
        grid=(V // VB,),
        in_specs=in_specs,
        out_specs=pl.BlockSpec((VB, OUT), lambda i: (i, 0)),
        out_shape=jax.ShapeDtypeStruct((V, OUT), jnp.float32),
    )(e0, e1, *lns, fcw, fcb)


NC, NS = 2, 16
NW = NC * NS
NROW = B // NW
C0, C1 = 128, L - 128


def _gather_body(table_hbm, x_hbm, out_hbm, idx_v, rows_v, sg0, sg1):
    wid = lax.axis_index("s") * NC + lax.axis_index("c")
    r0 = wid * NROW
    pltpu.sync_copy(x_hbm.at[pl.ds(r0, NROW)], idx_v)
    sgs = (sg0, sg1)

    def fire(r, bf):
        pltpu.async_copy(table_hbm.at[idx_v.at[r, pl.ds(0, C0)]],
                         rows_v.at[bf, pl.ds(0, C0)], sgs[bf])
        pltpu.async_copy(table_hbm.at[idx_v.at[r, pl.ds(C0, C1)]],
                         rows_v.at[bf, pl.ds(C0, C1)], sgs[bf])

    def wait(r, bf):
        pltpu.make_async_copy(table_hbm.at[idx_v.at[r, pl.ds(0, C0)]],
                              rows_v.at[bf, pl.ds(0, C0)], sgs[bf]).wait()
        pltpu.make_async_copy(table_hbm.at[idx_v.at[r, pl.ds(C0, C1)]],
                              rows_v.at[bf, pl.ds(C0, C1)], sgs[bf]).wait()

    def store(r, bf):
        pltpu.sync_copy(rows_v.at[bf], out_hbm.at[pl.ds((r0 + r) * L, L)])

    fire(0, 0)

    @pl.loop(0, NROW // 2 - 1)
    def _pair(ro):
        r = ro * 2
        fire(r + 1, 1)
        wait(r, 0)
        store(r, 0)
        fire(r + 2, 0)
        wait(r + 1, 1)
        store(r + 1, 1)

    fire(NROW - 1, 1)
    wait(NROW - 2, 0)
    store(NROW - 2, 0)
    wait(NROW - 1, 1)
    store(NROW - 1, 1)


@functools.lru_cache(maxsize=None)
def _get_gather():
    mesh = plsc.VectorSubcoreMesh(core_axis_name="c", subcore_axis_name="s",
                                  num_cores=NC, num_subcores=NS)
    return pl.kernel(
        _gather_body,
        out_type=jax.ShapeDtypeStruct((B * L, OUT), jnp.float32),
        mesh=mesh,
        scratch_types=[
            pltpu.VMEM((NROW, L), jnp.int32),
            pltpu.VMEM((2, L, OUT), jnp.float32),
            pltpu.SemaphoreType.DMA,
            pltpu.SemaphoreType.DMA,
        ],
        compiler_params=pltpu.CompilerParams(use_tc_tiling_on_sc=False),
    )


def kernel(x, emb0_w, emb1_w, g4, b4, g7, b7, g10, b10, g13, b13, g16, b16,
           fcW, fcb):
    lns = [a.reshape(1, D) for a in (g4, b4, g7, b7, g10, b10, g13, b13,
                                     g16, b16)]
    table = _build_table(emb0_w, emb1_w, lns, fcW, fcb.reshape(1, OUT))
    out = _get_gather()(table, x.astype(jnp.int32))
    return out.reshape(B, L, OUT)
